# baseline (speedup 1.0000x reference)
import functools

import jax
import jax.numpy as jnp
from jax import lax
from jax.experimental import pallas as pl
from jax.experimental.pallas import tpu as pltpu
from jax.experimental.pallas import tpu_sc as plsc

_LANES = 16
_NUM_WORKERS = 32
_NBUF = 4
_UNROLL = 6


def _build(batch, n, n_pad):
    rows_per_w = batch // _NUM_WORKERS
    num_groups = rows_per_w // _NBUF
    full_chunks = n // _LANES
    rem = n - full_chunks * _LANES
    bulk_end = 1 + ((full_chunks - 1) // _UNROLL) * _UNROLL
    mesh = plsc.VectorSubcoreMesh(core_axis_name="c", subcore_axis_name="s")

    @functools.partial(
        pl.kernel,
        mesh=mesh,
        out_type=jax.ShapeDtypeStruct((batch, n), jnp.float32),
        scratch_types=[
            pltpu.VMEM((1, n), jnp.float32),
            pltpu.VMEM((1, n), jnp.float32),
            pltpu.VMEM((1, n), jnp.float32),
            pltpu.VMEM((1, n), jnp.float32),
            pltpu.VMEM((n_pad,), jnp.int32),
            pltpu.SemaphoreType.DMA,
            pltpu.SemaphoreType.DMA,
            pltpu.SemaphoreType.DMA,
            pltpu.SemaphoreType.DMA,
            pltpu.SemaphoreType.DMA,
            pltpu.SemaphoreType.DMA,
            pltpu.SemaphoreType.DMA,
            pltpu.SemaphoreType.DMA,
        ],
        compiler_params=pltpu.CompilerParams(needs_layout_passes=False),
    )
    def run(p_hbm, idx_hbm, out_hbm, b0, b1, b2, b3,
            idx_v, si0, si1, si2, si3, so0, so1, so2, so3):
        bufs = (b0, b1, b2, b3)
        in_sems = (si0, si1, si2, si3)
        out_sems = (so0, so1, so2, so3)
        nc = 2
        wid = lax.axis_index("s") * nc + lax.axis_index("c")
        base = wid * rows_per_w
        lane = jax.lax.iota(jnp.int32, _LANES)
        zero16 = jnp.zeros((_LANES,), jnp.int32)

        pltpu.sync_copy(idx_hbm, idx_v)

        def in_cp(row, b):
            return pltpu.make_async_copy(
                p_hbm.at[pl.ds(row, 1)], bufs[b], in_sems[b])

        def out_cp(row, b):
            return pltpu.make_async_copy(
                bufs[b], out_hbm.at[pl.ds(row, 1)], out_sems[b])

        def compute(row_v):
            v0 = row_v[0, pl.ds(0, _LANES)]
            ix0 = idx_v[pl.ds(0, _LANES)]
            plsc.addupdate_scatter(row_v, [zero16, ix0], -v0, mask=lane >= 1)

            @plsc.parallel_loop(1, bulk_end, unroll=_UNROLL)
            def chunk_body(c):
                off = c * _LANES
                v = row_v[0, pl.ds(off, _LANES)]
                ix = idx_v[pl.ds(off, _LANES)]
                plsc.addupdate_scatter(row_v, [zero16, ix], -v)

            def tail_body(c, carry2):
                off = c * _LANES
                v = row_v[0, pl.ds(off, _LANES)]
                ix = idx_v[pl.ds(off, _LANES)]
                plsc.addupdate_scatter(row_v, [zero16, ix], -v)
                return carry2

            lax.fori_loop(bulk_end, full_chunks, tail_body, 0)

            if rem:
                off = full_chunks * _LANES
                col = jnp.minimum(off + lane, n - 1)
                vl = plsc.load_gather(row_v, [zero16, col], mask=lane < rem)
                ixl = idx_v[pl.ds(off, _LANES)]
                plsc.addupdate_scatter(
                    row_v, [zero16, ixl], -vl, mask=lane < rem)

        for g in range(3):
            in_cp(base + g, g).start()

        def step(g_row, b, *, prefetch, wait_prev_out):
            in_cp(g_row, b).wait()
            compute(bufs[b])
            out_cp(g_row, b).start()
            if wait_prev_out:
                out_cp(g_row - 1, (b + 3) % _NBUF).wait()
            if prefetch:
                in_cp(g_row + 3, (b + 3) % _NBUF).start()

        for j in range(_NBUF):
            step(base + j, j, prefetch=True, wait_prev_out=(j > 0))

        def group_body(go, carry):
            g0 = base + go * _NBUF
            for j in range(_NBUF):
                step(g0 + j, j, prefetch=True, wait_prev_out=True)
            return carry

        lax.fori_loop(1, num_groups - 1, group_body, 0)

        gl = base + (num_groups - 1) * _NBUF
        step(gl, 0, prefetch=True, wait_prev_out=True)
        for j in range(1, _NBUF):
            step(gl + j, j, prefetch=False, wait_prev_out=False)

        for j in range(_NBUF):
            out_cp(gl + j, j).wait()

    return run


def kernel(p, parent):
    batch, n = p.shape
    n_pad = ((n + 127) // 128) * 128
    idx = jnp.zeros((n_pad,), dtype=jnp.int32)
    idx = idx.at[1:n].set(parent[1:].astype(jnp.int32))
    run = _build(batch, n, n_pad)
    return run(p, idx)


# speedup vs baseline: 1.0055x; 1.0055x over previous
import functools

import jax
import jax.numpy as jnp
from jax import lax
from jax.experimental import pallas as pl
from jax.experimental.pallas import tpu as pltpu
from jax.experimental.pallas import tpu_sc as plsc

_LANES = 16
_NUM_WORKERS = 32
_NBUF = 4
_UNROLL = 4


def _build(batch, n, n_pad):
    rows_per_w = batch // _NUM_WORKERS
    num_groups = rows_per_w // _NBUF
    full_chunks = n // _LANES
    rem = n - full_chunks * _LANES
    bulk_end = 1 + ((full_chunks - 1) // _UNROLL) * _UNROLL
    mesh = plsc.VectorSubcoreMesh(core_axis_name="c", subcore_axis_name="s")

    @functools.partial(
        pl.kernel,
        mesh=mesh,
        out_type=jax.ShapeDtypeStruct((batch, n), jnp.float32),
        scratch_types=[
            pltpu.VMEM((1, n), jnp.float32),
            pltpu.VMEM((1, n), jnp.float32),
            pltpu.VMEM((1, n), jnp.float32),
            pltpu.VMEM((1, n), jnp.float32),
            pltpu.VMEM((n_pad,), jnp.int32),
            pltpu.SemaphoreType.DMA,
            pltpu.SemaphoreType.DMA,
            pltpu.SemaphoreType.DMA,
            pltpu.SemaphoreType.DMA,
            pltpu.SemaphoreType.DMA,
            pltpu.SemaphoreType.DMA,
            pltpu.SemaphoreType.DMA,
            pltpu.SemaphoreType.DMA,
        ],
        compiler_params=pltpu.CompilerParams(needs_layout_passes=False),
    )
    def run(p_hbm, idx_hbm, out_hbm, b0, b1, b2, b3,
            idx_v, si0, si1, si2, si3, so0, so1, so2, so3):
        bufs = (b0, b1, b2, b3)
        in_sems = (si0, si1, si2, si3)
        out_sems = (so0, so1, so2, so3)
        nc = 2
        wid = lax.axis_index("s") * nc + lax.axis_index("c")
        base = wid * rows_per_w
        lane = jax.lax.iota(jnp.int32, _LANES)
        zero16 = jnp.zeros((_LANES,), jnp.int32)

        pltpu.sync_copy(idx_hbm, idx_v)

        def in_cp(row, b):
            return pltpu.make_async_copy(
                p_hbm.at[pl.ds(row, 1)], bufs[b], in_sems[b])

        def out_cp(row, b):
            return pltpu.make_async_copy(
                bufs[b], out_hbm.at[pl.ds(row, 1)], out_sems[b])

        def compute(row_v):
            v0 = row_v[0, pl.ds(0, _LANES)]
            ix0 = idx_v[pl.ds(0, _LANES)]
            plsc.addupdate_scatter(row_v, [zero16, ix0], -v0, mask=lane >= 1)

            @plsc.parallel_loop(1, bulk_end, unroll=_UNROLL)
            def chunk_body(c):
                off = c * _LANES
                v = row_v[0, pl.ds(off, _LANES)]
                ix = idx_v[pl.ds(off, _LANES)]
                plsc.addupdate_scatter(row_v, [zero16, ix], -v)

            def tail_body(c, carry2):
                off = c * _LANES
                v = row_v[0, pl.ds(off, _LANES)]
                ix = idx_v[pl.ds(off, _LANES)]
                plsc.addupdate_scatter(row_v, [zero16, ix], -v)
                return carry2

            lax.fori_loop(bulk_end, full_chunks, tail_body, 0)

            if rem:
                off = full_chunks * _LANES
                col = jnp.minimum(off + lane, n - 1)
                vl = plsc.load_gather(row_v, [zero16, col], mask=lane < rem)
                ixl = idx_v[pl.ds(off, _LANES)]
                plsc.addupdate_scatter(
                    row_v, [zero16, ixl], -vl, mask=lane < rem)

        for g in range(3):
            in_cp(base + g, g).start()

        def step(g_row, b, *, prefetch, wait_prev_out):
            in_cp(g_row, b).wait()
            compute(bufs[b])
            out_cp(g_row, b).start()
            if wait_prev_out:
                out_cp(g_row - 1, (b + 3) % _NBUF).wait()
            if prefetch:
                in_cp(g_row + 3, (b + 3) % _NBUF).start()

        for j in range(_NBUF):
            step(base + j, j, prefetch=True, wait_prev_out=(j > 0))

        def group_body(go, carry):
            g0 = base + go * _NBUF
            for j in range(_NBUF):
                step(g0 + j, j, prefetch=True, wait_prev_out=True)
            return carry

        lax.fori_loop(1, num_groups - 1, group_body, 0)

        gl = base + (num_groups - 1) * _NBUF
        step(gl, 0, prefetch=True, wait_prev_out=True)
        for j in range(1, _NBUF):
            step(gl + j, j, prefetch=False, wait_prev_out=False)

        for j in range(_NBUF):
            out_cp(gl + j, j).wait()

    return run


def kernel(p, parent):
    batch, n = p.shape
    n_pad = ((n + 127) // 128) * 128
    idx = jnp.zeros((n_pad,), dtype=jnp.int32)
    idx = idx.at[1:n].set(parent[1:].astype(jnp.int32))
    run = _build(batch, n, n_pad)
    return run(p, idx)
